# baseline (speedup 1.0000x reference)
import functools

import jax
import jax.numpy as jnp
from jax import lax
from jax.experimental import pallas as pl
from jax.experimental.pallas import tpu as pltpu
from jax.experimental.pallas import tpu_sc as plsc

N = 10000
E = 160000
H = 256
HF = H // 2
NB = 5
EB = 2000
CH = 80
NSUB = 16
PER_TILE = E // NSUB
NPAD = 10240
NROWS = NPAD // NSUB
ZROWS = 128

_f32 = jnp.float32
_bf = jnp.bfloat16


def _edge_feat_body(el_ref, etr_ref, etp_ref, bond8_ref,
                    we1_ref, be1_ref, we2_ref, be2_ref,
                    wc1a_ref, wc1b_ref, bc1_ref, wc2_ref, bc2_ref,
                    wf11_ref, bf11_ref, wf21_ref, bf21_ref,
                    wf12_ref, bf12_ref, wf22_ref, bf22_ref,
                    wf13_ref, bf13_ref, wf23_ref, bf23_ref,
                    ea_ref, f1_ref, f2_ref, f3_ref):
    relu = jax.nn.relu
    el = jnp.sqrt(el_ref[...] + 1e-12)
    t = relu(el * we1_ref[...] + be1_ref[...]).astype(_bf)
    dist = jnp.dot(t, we2_ref[...], preferred_element_type=_f32) + be2_ref[...]
    iot = lax.broadcasted_iota(jnp.int32, (EB, 8), 1)
    ohr = (iot == etr_ref[...]).astype(_bf)
    ohp = (iot == etp_ref[...]).astype(_bf)
    br = jnp.dot(ohr, bond8_ref[...], preferred_element_type=_f32)
    bp = jnp.dot(ohp, bond8_ref[...], preferred_element_type=_f32)
    ear = (dist * br).astype(_bf)
    eap = (dist * bp).astype(_bf)
    t2 = relu(jnp.dot(ear, wc1a_ref[...], preferred_element_type=_f32)
              + jnp.dot(eap, wc1b_ref[...], preferred_element_type=_f32)
              + bc1_ref[...]).astype(_bf)
    ea = jnp.dot(t2, wc2_ref[...], preferred_element_type=_f32) + bc2_ref[...]
    eab = ea.astype(_bf)
    ea_ref[...] = eab
    for wf1, bf1, wf2, bf2, out in (
            (wf11_ref, bf11_ref, wf21_ref, bf21_ref, f1_ref),
            (wf12_ref, bf12_ref, wf22_ref, bf22_ref, f2_ref),
            (wf13_ref, bf13_ref, wf23_ref, bf23_ref, f3_ref)):
        h = relu(jnp.dot(eab, wf1[...], preferred_element_type=_f32)
                 + bf1[...]).astype(_bf)
        f = jnp.dot(h, wf2[...], preferred_element_type=_f32) + bf2[...]
        out[0] = f[:, :HF]
        out[1] = f[:, HF:]


def _full(shape):
    return pl.BlockSpec(shape, lambda i: (0,) * len(shape))


def _edge_features(el, etr, etp, bond8, params):
    grid = (E // EB,)
    eb_spec = pl.BlockSpec((EB, 1), lambda i: (i, 0))
    ea_spec = pl.BlockSpec((EB, H), lambda i: (i, 0))
    f_spec = pl.BlockSpec((2, EB, HF), lambda i: (0, i, 0))
    lys = params['layers']
    return pl.pallas_call(
        _edge_feat_body,
        grid=grid,
        in_specs=[eb_spec, eb_spec, eb_spec, _full((8, H)),
                  _full((1, H)), _full((1, H)), _full((H, H)), _full((1, H)),
                  _full((H, H)), _full((H, H)), _full((1, H)), _full((H, H)), _full((1, H))]
                 + [_full((H, H)), _full((1, H)), _full((H, H)), _full((1, H))] * 3,
        out_specs=[ea_spec, f_spec, f_spec, f_spec],
        out_shape=[jax.ShapeDtypeStruct((E, H), _bf)]
                  + [jax.ShapeDtypeStruct((2, E, HF), _f32)] * 3,
    )(el, etr, etp, bond8.astype(_bf),
      params['We1'], params['be1'].reshape(1, H),
      params['We2'].astype(_bf), params['be2'].reshape(1, H),
      params['Wc1'][:H].astype(_bf), params['Wc1'][H:].astype(_bf),
      params['bc1'].reshape(1, H),
      params['Wc2'].astype(_bf), params['bc2'].reshape(1, H),
      lys[0]['Wf1'].astype(_bf), lys[0]['bf1'].reshape(1, H),
      lys[0]['Wf2'].astype(_bf), lys[0]['bf2'].reshape(1, H),
      lys[1]['Wf1'].astype(_bf), lys[1]['bf1'].reshape(1, H),
      lys[1]['Wf2'].astype(_bf), lys[1]['bf2'].reshape(1, H),
      lys[2]['Wf1'].astype(_bf), lys[2]['bf1'].reshape(1, H),
      lys[2]['Wf2'].astype(_bf), lys[2]['bf2'].reshape(1, H))


_SC_MESH = plsc.VectorSubcoreMesh(core_axis_name="c", subcore_axis_name="s")
NCH = PER_TILE // CH


def _adjust_vec(buf, b, delta):
    for u in range(CH // 16):
        sl = pl.ds(u * 16, 16)
        buf[b, sl] = buf[b, sl] + delta


@functools.partial(
    pl.kernel,
    out_type=jax.ShapeDtypeStruct((2 * NPAD, HF), _f32),
    mesh=_SC_MESH,
    scratch_types=[
        pltpu.VMEM((2, CH), jnp.int32),
        pltpu.VMEM((2, CH), jnp.int32),
        pltpu.VMEM((2, CH, HF), _f32),
        pltpu.VMEM((2, CH, HF), _f32),
        pltpu.VMEM_SHARED((NPAD, HF), _f32),
        pltpu.SemaphoreType.DMA,
        pltpu.SemaphoreType.DMA,
        pltpu.SemaphoreType.DMA,
        pltpu.SemaphoreType.DMA,
        pltpu.SemaphoreType.DMA,
        pltpu.SemaphoreType.DMA,
        pltpu.SemaphoreType.DMA,
        pltpu.SemaphoreType.DMA,
    ],
)
def _sc_message_pass(xw_hbm, filt_hbm, row_hbm, col_hbm, out_hbm,
                     rowb, colb, gath, filtv, acc,
                     sr0, sr1, sc0, sc1, sg0, sg1, sf0, sf1):
    c = lax.axis_index("c")
    s = lax.axis_index("s")
    cN = c * N
    sri = (sr0, sr1)
    sci = (sc0, sc1)
    sg = (sg0, sg1)
    sf = (sf0, sf1)
    ebase = s * PER_TILE
    fbase = s * PER_TILE

    def zb(j, carry):
        for u in range(HF // 16):
            gath[0, j, pl.ds(u * 16, 16)] = jnp.zeros((16,), _f32)
        return carry

    lax.fori_loop(0, CH, zb, 0, unroll=False)
    for k in range(NROWS // CH):
        off = pl.multiple_of(s * NROWS + k * CH, 8)
        pltpu.sync_copy(gath.at[0], acc.at[pl.ds(off, CH)])
    plsc.subcore_barrier()

    def issue_idx(k, b):
        off = pl.multiple_of(ebase + k * CH, 8)
        pltpu.async_copy(row_hbm.at[pl.ds(off, CH)], rowb.at[b], sri[b])
        pltpu.async_copy(col_hbm.at[pl.ds(off, CH)], colb.at[b], sci[b])

    def issue_gather(k, b):
        pltpu.make_async_copy(row_hbm.at[pl.ds(0, CH)], rowb.at[b], sri[b]).wait()
        pltpu.make_async_copy(col_hbm.at[pl.ds(0, CH)], colb.at[b], sci[b]).wait()
        _adjust_vec(rowb, b, cN)
        pltpu.async_copy(xw_hbm.at[rowb.at[b]], gath.at[b], sg[b])
        pltpu.async_copy(filt_hbm.at[c, pl.ds(fbase + k * CH, CH)],
                         filtv.at[b], sf[b])

    def step(k, b):
        pltpu.make_async_copy(xw_hbm.at[pl.ds(0, CH)], gath.at[b], sg[b]).wait()
        pltpu.make_async_copy(filt_hbm.at[0, pl.ds(0, CH)], filtv.at[b], sf[b]).wait()

        @pl.when(k + 1 < NCH)
        def _():
            issue_gather(k + 1, 1 - b)

        def mulrow(j, cr):
            for u in range(HF // 16):
                sl = pl.ds(u * 16, 16)
                gath[b, j, sl] = gath[b, j, sl] * filtv[b, j, sl]
            return cr

        lax.fori_loop(0, CH, mulrow, 0, unroll=8)
        pltpu.sync_copy(gath.at[b], acc.at[colb.at[b]], add=True)

        @pl.when(k + 2 < NCH)
        def _():
            issue_idx(k + 2, b)

    issue_idx(0, 0)
    issue_idx(1, 1)
    issue_gather(0, 0)

    def pair(i, carry):
        step(2 * i, 0)
        step(2 * i + 1, 1)
        return carry

    lax.fori_loop(0, NCH // 2, pair, 0, unroll=False)
    step(NCH - 1, 0)

    plsc.subcore_barrier()
    src_off = pl.multiple_of(s * NROWS, 8)
    dst_off = pl.multiple_of(c * NPAD + s * NROWS, 8)
    pltpu.sync_copy(acc.at[pl.ds(src_off, NROWS)],
                    out_hbm.at[pl.ds(dst_off, NROWS)])


@functools.partial(
    pl.kernel,
    out_type=jax.ShapeDtypeStruct((2, E, HF), _f32),
    mesh=_SC_MESH,
    scratch_types=[
        pltpu.VMEM((2, CH), jnp.int32),
        pltpu.VMEM((2, CH), jnp.int32),
        pltpu.VMEM((2, CH, HF), _f32),
        pltpu.VMEM((2, CH, HF), _f32),
        pltpu.VMEM((2, CH, HF), _f32),
        pltpu.SemaphoreType.DMA,
        pltpu.SemaphoreType.DMA,
        pltpu.SemaphoreType.DMA,
        pltpu.SemaphoreType.DMA,
        pltpu.SemaphoreType.DMA,
        pltpu.SemaphoreType.DMA,
    ],
)
def _sc_pair_sum(x2_hbm, row_hbm, col_hbm, out_hbm,
                 rowb, colb, g1, g2, msum, sr0, sr1, sc0, sc1, sg0, sg1):
    c = lax.axis_index("c")
    s = lax.axis_index("s")
    cN = c * N
    sri = (sr0, sr1)
    sci = (sc0, sc1)
    sg = (sg0, sg1)
    ebase = s * PER_TILE
    obase = s * PER_TILE

    def issue_idx(k, b):
        off = pl.multiple_of(ebase + k * CH, 8)
        pltpu.async_copy(row_hbm.at[pl.ds(off, CH)], rowb.at[b], sri[b])
        pltpu.async_copy(col_hbm.at[pl.ds(off, CH)], colb.at[b], sci[b])

    def issue_gather(k, b):
        pltpu.make_async_copy(row_hbm.at[pl.ds(0, CH)], rowb.at[b], sri[b]).wait()
        pltpu.make_async_copy(col_hbm.at[pl.ds(0, CH)], colb.at[b], sci[b]).wait()
        _adjust_vec(rowb, b, cN)
        _adjust_vec(colb, b, cN)
        pltpu.async_copy(x2_hbm.at[rowb.at[b]], g1.at[b], sg[b])
        pltpu.async_copy(x2_hbm.at[colb.at[b]], g2.at[b], sg[b])

    def step(k, b):
        pltpu.make_async_copy(x2_hbm.at[pl.ds(0, CH)], g1.at[b], sg[b]).wait()
        pltpu.make_async_copy(x2_hbm.at[pl.ds(0, CH)], g2.at[b], sg[b]).wait()

        @pl.when(k + 1 < NCH)
        def _():
            issue_gather(k + 1, 1 - b)

        def addrow(j, cr):
            for u in range(HF // 16):
                sl = pl.ds(u * 16, 16)
                msum[b, j, sl] = g1[b, j, sl] + g2[b, j, sl]
            return cr

        lax.fori_loop(0, CH, addrow, 0, unroll=8)
        pltpu.sync_copy(msum.at[b], out_hbm.at[c, pl.ds(obase + k * CH, CH)])

        @pl.when(k + 2 < NCH)
        def _():
            issue_idx(k + 2, b)

    issue_idx(0, 0)
    issue_idx(1, 1)
    issue_gather(0, 0)

    def pair(i, carry):
        step(2 * i, 0)
        step(2 * i + 1, 1)
        return carry

    lax.fori_loop(0, NCH // 2, pair, 0, unroll=False)
    step(NCH - 1, 0)


EPW = E // 32


@functools.partial(
    pl.kernel,
    out_type=jax.ShapeDtypeStruct((E,), _f32),
    mesh=_SC_MESH,
    compiler_params=pltpu.CompilerParams(needs_layout_passes=False),
    scratch_types=[
        pltpu.VMEM((N * 8,), _f32),
        pltpu.VMEM((EPW,), jnp.int32),
        pltpu.VMEM((EPW,), jnp.int32),
        pltpu.VMEM((EPW,), _f32),
    ],
)
def _sc_edge_dist(pos_hbm, row_hbm, col_hbm, out_hbm, posv, rowv, colv, outv):
    c = lax.axis_index("c")
    s = lax.axis_index("s")
    w = s * 2 + c
    base = pl.multiple_of(w * EPW, 8)
    pltpu.sync_copy(pos_hbm, posv)
    pltpu.sync_copy(row_hbm.at[pl.ds(base, EPW)], rowv)
    pltpu.sync_copy(col_hbm.at[pl.ds(base, EPW)], colv)

    def grp_at(off):
        sl = pl.ds(off, 16)
        rv = rowv[sl] * 8
        cv = colv[sl] * 8
        acc = jnp.zeros((16,), _f32)
        for k in range(3):
            a = plsc.load_gather(posv, [rv + k])
            bvals = plsc.load_gather(posv, [cv + k])
            dk = a - bvals
            acc = acc + dk * dk
        outv[sl] = acc

    def grp(g, cr):
        grp_at(g * 16)
        return cr

    lax.fori_loop(0, EPW // 16, grp, 0, unroll=4)
    grp_at(EPW - 16)
    pltpu.sync_copy(outv, out_hbm.at[pl.ds(base, EPW)])


NBLK = 2000


def _node_init_body(at_ref, rf_ref, pf_ref, aemb_ref, waf_ref, wl1_ref,
                    x_ref, xw_ref):
    at = at_ref[...]
    oh_at = (lax.broadcasted_iota(jnp.int32, (NBLK, 104), 1) == at).astype(_f32)
    a_emb = jnp.dot(oh_at, aemb_ref[...], preferred_element_type=_f32)
    iot = lax.broadcasted_iota(jnp.int32, (NBLK, 16), 1)

    def feat(ref):
        r = ref[...]
        acc = jnp.zeros((NBLK, HF), _f32)
        for j in range(3):
            oh = (iot == r[:, j:j + 1]).astype(_f32)
            acc = acc + jnp.dot(oh, waf_ref[pl.ds(j * 16, 16), :],
                                preferred_element_type=_f32)
        return acc

    af_r = feat(rf_ref)
    af_p = feat(pf_ref)
    x = jnp.concatenate([a_emb + af_r, af_p - af_r], axis=-1)
    x_ref[...] = x
    proj = jnp.dot(x, wl1_ref[...], preferred_element_type=_f32)
    xw_ref[0] = proj[:, :HF]
    xw_ref[1] = proj[:, HF:]


def _node_init(atom_type, r_feat, p_feat, aemb_pad, waf_pad, wl1):
    grid = (N // NBLK,)
    return pl.pallas_call(
        _node_init_body,
        grid=grid,
        in_specs=[pl.BlockSpec((NBLK, 1), lambda i: (i, 0)),
                  pl.BlockSpec((NBLK, 3), lambda i: (i, 0)),
                  pl.BlockSpec((NBLK, 3), lambda i: (i, 0)),
                  _full((104, HF)), _full((48, HF)), _full((H, H))],
        out_specs=[pl.BlockSpec((NBLK, H), lambda i: (i, 0)),
                   pl.BlockSpec((2, NBLK, HF), lambda i: (0, i, 0))],
        out_shape=[jax.ShapeDtypeStruct((N, H), _f32),
                   jax.ShapeDtypeStruct((2, N, HF), _f32)],
    )(atom_type, r_feat, p_feat, aemb_pad, waf_pad, wl1)


def _layer_update_body(x_ref, agg_ref, wl2_ref, bl2_ref, wl1n_ref,
                       xn_ref, xw_ref):
    agg = jnp.concatenate([agg_ref[0], agg_ref[1]], axis=-1)
    xn = (x_ref[...] + jnp.dot(jax.nn.relu(agg), wl2_ref[...],
                               preferred_element_type=_f32) + bl2_ref[...])
    xn_ref[...] = xn
    proj = jnp.dot(xn, wl1n_ref[...], preferred_element_type=_f32)
    xw_ref[0] = proj[:, :HF]
    xw_ref[1] = proj[:, HF:]


def _layer_update_last_body(x_ref, agg_ref, wl2_ref, bl2_ref, x2_ref):
    agg = jnp.concatenate([agg_ref[0], agg_ref[1]], axis=-1)
    xn = (x_ref[...] + jnp.dot(jax.nn.relu(agg), wl2_ref[...],
                               preferred_element_type=_f32) + bl2_ref[...])
    x2_ref[0] = xn[:, :HF]
    x2_ref[1] = xn[:, HF:]


def _layer_update(x, agg2, wl2, bl2, wl1n):
    grid = (N // NBLK,)
    agg3 = agg2.reshape(2, NPAD, HF)
    return pl.pallas_call(
        _layer_update_body,
        grid=grid,
        in_specs=[pl.BlockSpec((NBLK, H), lambda i: (i, 0)),
                  pl.BlockSpec((2, NBLK, HF), lambda i: (0, i, 0)),
                  _full((H, H)), _full((1, H)), _full((H, H))],
        out_specs=[pl.BlockSpec((NBLK, H), lambda i: (i, 0)),
                   pl.BlockSpec((2, NBLK, HF), lambda i: (0, i, 0))],
        out_shape=[jax.ShapeDtypeStruct((N, H), _f32),
                   jax.ShapeDtypeStruct((2, N, HF), _f32)],
    )(x, agg3, wl2, bl2.reshape(1, H), wl1n)


def _layer_update_last(x, agg2, wl2, bl2):
    grid = (N // NBLK,)
    agg3 = agg2.reshape(2, NPAD, HF)
    return pl.pallas_call(
        _layer_update_last_body,
        grid=grid,
        in_specs=[pl.BlockSpec((NBLK, H), lambda i: (i, 0)),
                  pl.BlockSpec((2, NBLK, HF), lambda i: (0, i, 0)),
                  _full((H, H)), _full((1, H))],
        out_specs=pl.BlockSpec((2, NBLK, HF), lambda i: (0, i, 0)),
        out_shape=jax.ShapeDtypeStruct((2, N, HF), _f32),
    )(x, agg3, wl2, bl2.reshape(1, H))


def _final_body(hsum_ref, ea_ref, mask_ref,
                wg1a_ref, wg1b_ref, bg1_ref, wg2_ref, bg2_ref, wg3_ref, bg3_ref,
                out_ref):
    relu = jax.nn.relu
    hs = jnp.concatenate([hsum_ref[0], hsum_ref[1]], axis=-1).astype(_bf)
    h1 = relu(jnp.dot(hs, wg1a_ref[...], preferred_element_type=_f32)
              + jnp.dot(ea_ref[...], wg1b_ref[...], preferred_element_type=_f32)
              + bg1_ref[...]).astype(_bf)
    h2 = relu(jnp.dot(h1, wg2_ref[...], preferred_element_type=_f32) + bg2_ref[...])
    o = jnp.dot(h2, wg3_ref[...], preferred_element_type=_f32) + bg3_ref[...]
    out_ref[...] = o * mask_ref[...]


def _final_mlp(hsum2, ea, mask, params):
    grid = (E // EB,)
    hs_spec = pl.BlockSpec((2, EB, HF), lambda i: (0, i, 0))
    big = pl.BlockSpec((EB, H), lambda i: (i, 0))
    one = pl.BlockSpec((EB, 1), lambda i: (i, 0))
    return pl.pallas_call(
        _final_body,
        grid=grid,
        in_specs=[hs_spec, big, one,
                  _full((H, H)), _full((H, H)), _full((1, H)),
                  _full((H, HF)), _full((1, HF)), _full((HF, 1)), _full((1, 1))],
        out_specs=one,
        out_shape=jax.ShapeDtypeStruct((E, 1), _f32),
    )(hsum2, ea, mask,
      params['Wg1'][:H].astype(_bf), params['Wg1'][H:].astype(_bf),
      params['bg1'].reshape(1, H),
      params['Wg2'].astype(_bf), params['bg2'].reshape(1, HF),
      params['Wg3'], params['bg3'].reshape(1, 1))


def kernel(atom_type, r_feat, p_feat, pos, bond_index, bond_type, batch, params):
    lys = params['layers']
    row = bond_index[0].astype(jnp.int32)
    col = bond_index[1].astype(jnp.int32)
    pos_pad = jnp.pad(pos, ((0, 0), (0, 5))).reshape(-1)
    d2 = _sc_edge_dist(pos_pad, row, col)

    bt = bond_type.astype(jnp.int32)
    etr = (bt // NB).reshape(E, 1)
    etp = (bt % NB).reshape(E, 1)
    bond8 = params['bond_emb'][:8]

    ea, f1, f2, f3 = _edge_features(d2.reshape(E, 1), etr, etp, bond8, params)

    aemb_pad = jnp.pad(params['atom_emb'], ((0, 4), (0, 0)))
    waf_pad = jnp.pad(params['W_af'].reshape(3, 10, HF),
                      ((0, 0), (0, 6), (0, 0))).reshape(48, HF)
    x, xw2 = _node_init(atom_type.reshape(N, 1).astype(jnp.int32),
                        r_feat.astype(jnp.int32), p_feat.astype(jnp.int32),
                        aemb_pad, waf_pad, lys[0]['Wl1'])

    for l, filt in enumerate((f1, f2, f3)):
        agg2 = _sc_message_pass(xw2.reshape(2 * N, HF), filt, row, col)
        if l < 2:
            x, xw2 = _layer_update(x, agg2, lys[l]['Wl2'], lys[l]['bl2'],
                                   lys[l + 1]['Wl1'])
        else:
            x2 = _layer_update_last(x, agg2, lys[l]['Wl2'], lys[l]['bl2'])

    hsum2 = _sc_pair_sum(x2.reshape(2 * N, HF), row, col)
    mask = (row < col).astype(_f32).reshape(E, 1)
    return _final_mlp(hsum2, ea, mask, params)


# speedup vs baseline: 1.0154x; 1.0154x over previous
import functools

import jax
import jax.numpy as jnp
from jax import lax
from jax.experimental import pallas as pl
from jax.experimental.pallas import tpu as pltpu
from jax.experimental.pallas import tpu_sc as plsc

N = 10000
E = 160000
H = 256
HF = H // 2
NB = 5
EB = 2000
CH = 80
NSUB = 16
PER_TILE = E // NSUB
NPAD = 10240
NROWS = NPAD // NSUB
ZROWS = 128

_f32 = jnp.float32
_bf = jnp.bfloat16


def _edge_feat_body(el_ref, etr_ref, etp_ref, bond8_ref,
                    we1_ref, be1_ref, we2_ref, be2_ref,
                    wc1a_ref, wc1b_ref, bc1_ref, wc2_ref, bc2_ref,
                    wf11_ref, bf11_ref, wf21_ref, bf21_ref,
                    wf12_ref, bf12_ref, wf22_ref, bf22_ref,
                    wf13_ref, bf13_ref, wf23_ref, bf23_ref,
                    ea_ref, f1_ref, f2_ref, f3_ref):
    relu = jax.nn.relu
    el = jnp.sqrt(el_ref[...] + 1e-12)
    t = relu(el * we1_ref[...] + be1_ref[...]).astype(_bf)
    dist = jnp.dot(t, we2_ref[...], preferred_element_type=_f32) + be2_ref[...]
    iot = lax.broadcasted_iota(jnp.int32, (EB, 8), 1)
    ohr = (iot == etr_ref[...]).astype(_bf)
    ohp = (iot == etp_ref[...]).astype(_bf)
    br = jnp.dot(ohr, bond8_ref[...], preferred_element_type=_f32)
    bp = jnp.dot(ohp, bond8_ref[...], preferred_element_type=_f32)
    ear = (dist * br).astype(_bf)
    eap = (dist * bp).astype(_bf)
    t2 = relu(jnp.dot(ear, wc1a_ref[...], preferred_element_type=_f32)
              + jnp.dot(eap, wc1b_ref[...], preferred_element_type=_f32)
              + bc1_ref[...]).astype(_bf)
    ea = jnp.dot(t2, wc2_ref[...], preferred_element_type=_f32) + bc2_ref[...]
    eab = ea.astype(_bf)
    ea_ref[...] = eab
    for wf1, bf1, wf2, bf2, out in (
            (wf11_ref, bf11_ref, wf21_ref, bf21_ref, f1_ref),
            (wf12_ref, bf12_ref, wf22_ref, bf22_ref, f2_ref),
            (wf13_ref, bf13_ref, wf23_ref, bf23_ref, f3_ref)):
        h = relu(jnp.dot(eab, wf1[...], preferred_element_type=_f32)
                 + bf1[...]).astype(_bf)
        f = jnp.dot(h, wf2[...], preferred_element_type=_f32) + bf2[...]
        out[0] = f[:, :HF]
        out[1] = f[:, HF:]


def _full(shape):
    return pl.BlockSpec(shape, lambda i: (0,) * len(shape))


def _edge_features(el, etr, etp, bond8, params):
    grid = (E // EB,)
    eb_spec = pl.BlockSpec((EB, 1), lambda i: (i, 0))
    ea_spec = pl.BlockSpec((EB, H), lambda i: (i, 0))
    f_spec = pl.BlockSpec((2, EB, HF), lambda i: (0, i, 0))
    lys = params['layers']
    return pl.pallas_call(
        _edge_feat_body,
        grid=grid,
        in_specs=[eb_spec, eb_spec, eb_spec, _full((8, H)),
                  _full((1, H)), _full((1, H)), _full((H, H)), _full((1, H)),
                  _full((H, H)), _full((H, H)), _full((1, H)), _full((H, H)), _full((1, H))]
                 + [_full((H, H)), _full((1, H)), _full((H, H)), _full((1, H))] * 3,
        out_specs=[ea_spec, f_spec, f_spec, f_spec],
        out_shape=[jax.ShapeDtypeStruct((E, H), _bf)]
                  + [jax.ShapeDtypeStruct((2, E, HF), _f32)] * 3,
    )(el, etr, etp, bond8.astype(_bf),
      params['We1'], params['be1'].reshape(1, H),
      params['We2'].astype(_bf), params['be2'].reshape(1, H),
      params['Wc1'][:H].astype(_bf), params['Wc1'][H:].astype(_bf),
      params['bc1'].reshape(1, H),
      params['Wc2'].astype(_bf), params['bc2'].reshape(1, H),
      lys[0]['Wf1'].astype(_bf), lys[0]['bf1'].reshape(1, H),
      lys[0]['Wf2'].astype(_bf), lys[0]['bf2'].reshape(1, H),
      lys[1]['Wf1'].astype(_bf), lys[1]['bf1'].reshape(1, H),
      lys[1]['Wf2'].astype(_bf), lys[1]['bf2'].reshape(1, H),
      lys[2]['Wf1'].astype(_bf), lys[2]['bf1'].reshape(1, H),
      lys[2]['Wf2'].astype(_bf), lys[2]['bf2'].reshape(1, H))


_SC_MESH = plsc.VectorSubcoreMesh(core_axis_name="c", subcore_axis_name="s")
NCH = PER_TILE // CH


def _adjust_vec(buf, b, delta):
    for u in range(CH // 16):
        sl = pl.ds(u * 16, 16)
        buf[b, sl] = buf[b, sl] + delta


@functools.partial(
    pl.kernel,
    out_type=jax.ShapeDtypeStruct((2 * NPAD, HF), _f32),
    mesh=_SC_MESH,
    scratch_types=[
        pltpu.VMEM((2, CH), jnp.int32),
        pltpu.VMEM((2, CH), jnp.int32),
        pltpu.VMEM((2, CH, HF), _f32),
        pltpu.VMEM((2, CH, HF), _f32),
        pltpu.VMEM_SHARED((NPAD, HF), _f32),
        pltpu.SemaphoreType.DMA,
        pltpu.SemaphoreType.DMA,
        pltpu.SemaphoreType.DMA,
        pltpu.SemaphoreType.DMA,
        pltpu.SemaphoreType.DMA,
        pltpu.SemaphoreType.DMA,
        pltpu.SemaphoreType.DMA,
        pltpu.SemaphoreType.DMA,
    ],
)
def _sc_message_pass(xw_hbm, filt_hbm, row_hbm, col_hbm, out_hbm,
                     rowb, colb, gath, filtv, acc,
                     sr0, sr1, sc0, sc1, sg0, sg1, sf0, sf1):
    c = lax.axis_index("c")
    s = lax.axis_index("s")
    cN = c * N
    sri = (sr0, sr1)
    sci = (sc0, sc1)
    sg = (sg0, sg1)
    sf = (sf0, sf1)
    ebase = s * PER_TILE
    fbase = s * PER_TILE

    def zb(j, carry):
        for u in range(HF // 16):
            gath[0, j, pl.ds(u * 16, 16)] = jnp.zeros((16,), _f32)
        return carry

    lax.fori_loop(0, CH, zb, 0, unroll=False)
    for k in range(NROWS // CH):
        off = pl.multiple_of(s * NROWS + k * CH, 8)
        pltpu.sync_copy(gath.at[0], acc.at[pl.ds(off, CH)])
    plsc.subcore_barrier()

    def issue_idx(k, b):
        off = pl.multiple_of(ebase + k * CH, 8)
        pltpu.async_copy(row_hbm.at[pl.ds(off, CH)], rowb.at[b], sri[b])
        pltpu.async_copy(col_hbm.at[pl.ds(off, CH)], colb.at[b], sci[b])

    def issue_gather(k, b):
        pltpu.make_async_copy(row_hbm.at[pl.ds(0, CH)], rowb.at[b], sri[b]).wait()
        pltpu.make_async_copy(col_hbm.at[pl.ds(0, CH)], colb.at[b], sci[b]).wait()
        _adjust_vec(rowb, b, cN)
        pltpu.async_copy(xw_hbm.at[rowb.at[b]], gath.at[b], sg[b])
        pltpu.async_copy(filt_hbm.at[c, pl.ds(fbase + k * CH, CH)],
                         filtv.at[b], sf[b])

    def step(k, b):
        pltpu.make_async_copy(xw_hbm.at[pl.ds(0, CH)], gath.at[b], sg[b]).wait()
        pltpu.make_async_copy(filt_hbm.at[0, pl.ds(0, CH)], filtv.at[b], sf[b]).wait()

        @pl.when(k + 1 < NCH)
        def _():
            issue_gather(k + 1, 1 - b)

        def mulrow(j, cr):
            for u in range(HF // 16):
                sl = pl.ds(u * 16, 16)
                gath[b, j, sl] = gath[b, j, sl] * filtv[b, j, sl]
            return cr

        lax.fori_loop(0, CH, mulrow, 0, unroll=8)
        pltpu.sync_copy(gath.at[b], acc.at[colb.at[b]], add=True)

        @pl.when(k + 2 < NCH)
        def _():
            issue_idx(k + 2, b)

    issue_idx(0, 0)
    issue_idx(1, 1)
    issue_gather(0, 0)

    def pair(i, carry):
        step(2 * i, 0)
        step(2 * i + 1, 1)
        return carry

    lax.fori_loop(0, NCH // 2, pair, 0, unroll=False)
    step(NCH - 1, 0)

    plsc.subcore_barrier()
    src_off = pl.multiple_of(s * NROWS, 8)
    dst_off = pl.multiple_of(c * NPAD + s * NROWS, 8)
    pltpu.sync_copy(acc.at[pl.ds(src_off, NROWS)],
                    out_hbm.at[pl.ds(dst_off, NROWS)])


@functools.partial(
    pl.kernel,
    out_type=jax.ShapeDtypeStruct((2, E, HF), _f32),
    mesh=_SC_MESH,
    scratch_types=[
        pltpu.VMEM((2, CH), jnp.int32),
        pltpu.VMEM((2, CH), jnp.int32),
        pltpu.VMEM((2, CH, HF), _f32),
        pltpu.VMEM((2, CH, HF), _f32),
        pltpu.SemaphoreType.DMA,
        pltpu.SemaphoreType.DMA,
        pltpu.SemaphoreType.DMA,
        pltpu.SemaphoreType.DMA,
        pltpu.SemaphoreType.DMA,
        pltpu.SemaphoreType.DMA,
    ],
)
def _sc_pair_sum(x2_hbm, row_hbm, col_hbm, out_hbm,
                 rowb, colb, g1, g2, sr0, sr1, sc0, sc1, sg0, sg1):
    c = lax.axis_index("c")
    s = lax.axis_index("s")
    cN = c * N
    sri = (sr0, sr1)
    sci = (sc0, sc1)
    sg = (sg0, sg1)
    ebase = s * PER_TILE
    obase = s * PER_TILE

    def issue_idx(k, b):
        off = pl.multiple_of(ebase + k * CH, 8)
        pltpu.async_copy(row_hbm.at[pl.ds(off, CH)], rowb.at[b], sri[b])
        pltpu.async_copy(col_hbm.at[pl.ds(off, CH)], colb.at[b], sci[b])

    def issue_gather(k, b):
        pltpu.make_async_copy(row_hbm.at[pl.ds(0, CH)], rowb.at[b], sri[b]).wait()
        pltpu.make_async_copy(col_hbm.at[pl.ds(0, CH)], colb.at[b], sci[b]).wait()
        _adjust_vec(rowb, b, cN)
        _adjust_vec(colb, b, cN)
        pltpu.async_copy(x2_hbm.at[rowb.at[b]], g1.at[b], sg[b])
        pltpu.async_copy(x2_hbm.at[colb.at[b]], g2.at[b], sg[b])

    def step(k, b):
        pltpu.make_async_copy(x2_hbm.at[pl.ds(0, CH)], g1.at[b], sg[b]).wait()
        pltpu.make_async_copy(x2_hbm.at[pl.ds(0, CH)], g2.at[b], sg[b]).wait()

        @pl.when(k + 1 < NCH)
        def _():
            issue_gather(k + 1, 1 - b)

        def addrow(j, cr):
            for u in range(HF // 16):
                sl = pl.ds(u * 16, 16)
                g1[b, j, sl] = g1[b, j, sl] + g2[b, j, sl]
            return cr

        lax.fori_loop(0, CH, addrow, 0, unroll=8)
        pltpu.sync_copy(g1.at[b], out_hbm.at[c, pl.ds(obase + k * CH, CH)])

        @pl.when(k + 2 < NCH)
        def _():
            issue_idx(k + 2, b)

    issue_idx(0, 0)
    issue_idx(1, 1)
    issue_gather(0, 0)

    def pair(i, carry):
        step(2 * i, 0)
        step(2 * i + 1, 1)
        return carry

    lax.fori_loop(0, NCH // 2, pair, 0, unroll=False)
    step(NCH - 1, 0)


EPW = E // 32


@functools.partial(
    pl.kernel,
    out_type=jax.ShapeDtypeStruct((E,), _f32),
    mesh=_SC_MESH,
    compiler_params=pltpu.CompilerParams(needs_layout_passes=False),
    scratch_types=[
        pltpu.VMEM((N * 8,), _f32),
        pltpu.VMEM((EPW,), jnp.int32),
        pltpu.VMEM((EPW,), jnp.int32),
        pltpu.VMEM((EPW,), _f32),
    ],
)
def _sc_edge_dist(pos_hbm, row_hbm, col_hbm, out_hbm, posv, rowv, colv, outv):
    c = lax.axis_index("c")
    s = lax.axis_index("s")
    w = s * 2 + c
    base = pl.multiple_of(w * EPW, 8)
    pltpu.sync_copy(pos_hbm, posv)
    pltpu.sync_copy(row_hbm.at[pl.ds(base, EPW)], rowv)
    pltpu.sync_copy(col_hbm.at[pl.ds(base, EPW)], colv)

    def grp_at(off):
        sl = pl.ds(off, 16)
        rv = rowv[sl] * 8
        cv = colv[sl] * 8
        acc = jnp.zeros((16,), _f32)
        for k in range(3):
            a = plsc.load_gather(posv, [rv + k])
            bvals = plsc.load_gather(posv, [cv + k])
            dk = a - bvals
            acc = acc + dk * dk
        outv[sl] = acc

    def grp(g, cr):
        grp_at(g * 16)
        return cr

    lax.fori_loop(0, EPW // 16, grp, 0, unroll=4)
    grp_at(EPW - 16)
    pltpu.sync_copy(outv, out_hbm.at[pl.ds(base, EPW)])


NBLK = 2000


def _node_init_body(at_ref, rf_ref, pf_ref, aemb_ref, waf_ref, wl1_ref,
                    x_ref, xw_ref):
    at = at_ref[...]
    oh_at = (lax.broadcasted_iota(jnp.int32, (NBLK, 104), 1) == at).astype(_f32)
    a_emb = jnp.dot(oh_at, aemb_ref[...], preferred_element_type=_f32)
    iot = lax.broadcasted_iota(jnp.int32, (NBLK, 16), 1)

    def feat(ref):
        r = ref[...]
        acc = jnp.zeros((NBLK, HF), _f32)
        for j in range(3):
            oh = (iot == r[:, j:j + 1]).astype(_f32)
            acc = acc + jnp.dot(oh, waf_ref[pl.ds(j * 16, 16), :],
                                preferred_element_type=_f32)
        return acc

    af_r = feat(rf_ref)
    af_p = feat(pf_ref)
    x = jnp.concatenate([a_emb + af_r, af_p - af_r], axis=-1)
    x_ref[...] = x
    proj = jnp.dot(x, wl1_ref[...], preferred_element_type=_f32)
    xw_ref[0] = proj[:, :HF]
    xw_ref[1] = proj[:, HF:]


def _node_init(atom_type, r_feat, p_feat, aemb_pad, waf_pad, wl1):
    grid = (N // NBLK,)
    return pl.pallas_call(
        _node_init_body,
        grid=grid,
        in_specs=[pl.BlockSpec((NBLK, 1), lambda i: (i, 0)),
                  pl.BlockSpec((NBLK, 3), lambda i: (i, 0)),
                  pl.BlockSpec((NBLK, 3), lambda i: (i, 0)),
                  _full((104, HF)), _full((48, HF)), _full((H, H))],
        out_specs=[pl.BlockSpec((NBLK, H), lambda i: (i, 0)),
                   pl.BlockSpec((2, NBLK, HF), lambda i: (0, i, 0))],
        out_shape=[jax.ShapeDtypeStruct((N, H), _f32),
                   jax.ShapeDtypeStruct((2, N, HF), _f32)],
    )(atom_type, r_feat, p_feat, aemb_pad, waf_pad, wl1)


def _layer_update_body(x_ref, agg_ref, wl2_ref, bl2_ref, wl1n_ref,
                       xn_ref, xw_ref):
    agg = jnp.concatenate([agg_ref[0], agg_ref[1]], axis=-1)
    xn = (x_ref[...] + jnp.dot(jax.nn.relu(agg), wl2_ref[...],
                               preferred_element_type=_f32) + bl2_ref[...])
    xn_ref[...] = xn
    proj = jnp.dot(xn, wl1n_ref[...], preferred_element_type=_f32)
    xw_ref[0] = proj[:, :HF]
    xw_ref[1] = proj[:, HF:]


def _layer_update_last_body(x_ref, agg_ref, wl2_ref, bl2_ref, x2_ref):
    agg = jnp.concatenate([agg_ref[0], agg_ref[1]], axis=-1)
    xn = (x_ref[...] + jnp.dot(jax.nn.relu(agg), wl2_ref[...],
                               preferred_element_type=_f32) + bl2_ref[...])
    x2_ref[0] = xn[:, :HF]
    x2_ref[1] = xn[:, HF:]


def _layer_update(x, agg2, wl2, bl2, wl1n):
    grid = (N // NBLK,)
    agg3 = agg2.reshape(2, NPAD, HF)
    return pl.pallas_call(
        _layer_update_body,
        grid=grid,
        in_specs=[pl.BlockSpec((NBLK, H), lambda i: (i, 0)),
                  pl.BlockSpec((2, NBLK, HF), lambda i: (0, i, 0)),
                  _full((H, H)), _full((1, H)), _full((H, H))],
        out_specs=[pl.BlockSpec((NBLK, H), lambda i: (i, 0)),
                   pl.BlockSpec((2, NBLK, HF), lambda i: (0, i, 0))],
        out_shape=[jax.ShapeDtypeStruct((N, H), _f32),
                   jax.ShapeDtypeStruct((2, N, HF), _f32)],
    )(x, agg3, wl2, bl2.reshape(1, H), wl1n)


def _layer_update_last(x, agg2, wl2, bl2):
    grid = (N // NBLK,)
    agg3 = agg2.reshape(2, NPAD, HF)
    return pl.pallas_call(
        _layer_update_last_body,
        grid=grid,
        in_specs=[pl.BlockSpec((NBLK, H), lambda i: (i, 0)),
                  pl.BlockSpec((2, NBLK, HF), lambda i: (0, i, 0)),
                  _full((H, H)), _full((1, H))],
        out_specs=pl.BlockSpec((2, NBLK, HF), lambda i: (0, i, 0)),
        out_shape=jax.ShapeDtypeStruct((2, N, HF), _f32),
    )(x, agg3, wl2, bl2.reshape(1, H))


def _final_body(hsum_ref, ea_ref, mask_ref,
                wg1a_ref, wg1b_ref, bg1_ref, wg2_ref, bg2_ref, wg3_ref, bg3_ref,
                out_ref):
    relu = jax.nn.relu
    hs = jnp.concatenate([hsum_ref[0], hsum_ref[1]], axis=-1).astype(_bf)
    h1 = relu(jnp.dot(hs, wg1a_ref[...], preferred_element_type=_f32)
              + jnp.dot(ea_ref[...], wg1b_ref[...], preferred_element_type=_f32)
              + bg1_ref[...]).astype(_bf)
    h2 = relu(jnp.dot(h1, wg2_ref[...], preferred_element_type=_f32) + bg2_ref[...])
    o = jnp.dot(h2, wg3_ref[...], preferred_element_type=_f32) + bg3_ref[...]
    out_ref[...] = o * mask_ref[...]


def _final_mlp(hsum2, ea, mask, params):
    grid = (E // EB,)
    hs_spec = pl.BlockSpec((2, EB, HF), lambda i: (0, i, 0))
    big = pl.BlockSpec((EB, H), lambda i: (i, 0))
    one = pl.BlockSpec((EB, 1), lambda i: (i, 0))
    return pl.pallas_call(
        _final_body,
        grid=grid,
        in_specs=[hs_spec, big, one,
                  _full((H, H)), _full((H, H)), _full((1, H)),
                  _full((H, HF)), _full((1, HF)), _full((HF, 1)), _full((1, 1))],
        out_specs=one,
        out_shape=jax.ShapeDtypeStruct((E, 1), _f32),
    )(hsum2, ea, mask,
      params['Wg1'][:H].astype(_bf), params['Wg1'][H:].astype(_bf),
      params['bg1'].reshape(1, H),
      params['Wg2'].astype(_bf), params['bg2'].reshape(1, HF),
      params['Wg3'], params['bg3'].reshape(1, 1))


def kernel(atom_type, r_feat, p_feat, pos, bond_index, bond_type, batch, params):
    lys = params['layers']
    row = bond_index[0].astype(jnp.int32)
    col = bond_index[1].astype(jnp.int32)
    pos_pad = jnp.pad(pos, ((0, 0), (0, 5))).reshape(-1)
    d2 = _sc_edge_dist(pos_pad, row, col)

    bt = bond_type.astype(jnp.int32)
    etr = (bt // NB).reshape(E, 1)
    etp = (bt % NB).reshape(E, 1)
    bond8 = params['bond_emb'][:8]

    ea, f1, f2, f3 = _edge_features(d2.reshape(E, 1), etr, etp, bond8, params)

    aemb_pad = jnp.pad(params['atom_emb'], ((0, 4), (0, 0)))
    waf_pad = jnp.pad(params['W_af'].reshape(3, 10, HF),
                      ((0, 0), (0, 6), (0, 0))).reshape(48, HF)
    x, xw2 = _node_init(atom_type.reshape(N, 1).astype(jnp.int32),
                        r_feat.astype(jnp.int32), p_feat.astype(jnp.int32),
                        aemb_pad, waf_pad, lys[0]['Wl1'])

    for l, filt in enumerate((f1, f2, f3)):
        agg2 = _sc_message_pass(xw2.reshape(2 * N, HF), filt, row, col)
        if l < 2:
            x, xw2 = _layer_update(x, agg2, lys[l]['Wl2'], lys[l]['bl2'],
                                   lys[l + 1]['Wl1'])
        else:
            x2 = _layer_update_last(x, agg2, lys[l]['Wl2'], lys[l]['bl2'])

    hsum2 = _sc_pair_sum(x2.reshape(2 * N, HF), row, col)
    mask = (row < col).astype(_f32).reshape(E, 1)
    return _final_mlp(hsum2, ea, mask, params)


# speedup vs baseline: 1.6161x; 1.5916x over previous
import functools

import jax
import jax.numpy as jnp
from jax import lax
from jax.experimental import pallas as pl
from jax.experimental.pallas import tpu as pltpu
from jax.experimental.pallas import tpu_sc as plsc

N = 10000
E = 160000
H = 256
HF = H // 2
NB = 5
EB = 2000
CH = 80
NSUB = 16
PER_TILE = E // NSUB
NPAD = 10240
NROWS = NPAD // NSUB
ZROWS = 128

_f32 = jnp.float32
_bf = jnp.bfloat16


def _edge_feat_body(el_ref, etr_ref, etp_ref, bond8_ref,
                    we1_ref, be1_ref, we2_ref, be2_ref,
                    wc1a_ref, wc1b_ref, bc1_ref, wc2_ref, bc2_ref,
                    wf11_ref, bf11_ref, wf21_ref, bf21_ref,
                    wf12_ref, bf12_ref, wf22_ref, bf22_ref,
                    wf13_ref, bf13_ref, wf23_ref, bf23_ref,
                    ea_ref, f1_ref, f2_ref, f3_ref):
    relu = jax.nn.relu
    el = jnp.sqrt(el_ref[...] + 1e-12)
    t = relu(el * we1_ref[...] + be1_ref[...]).astype(_bf)
    dist = jnp.dot(t, we2_ref[...], preferred_element_type=_f32) + be2_ref[...]
    iot = lax.broadcasted_iota(jnp.int32, (EB, 8), 1)
    ohr = (iot == etr_ref[...]).astype(_bf)
    ohp = (iot == etp_ref[...]).astype(_bf)
    br = jnp.dot(ohr, bond8_ref[...], preferred_element_type=_f32)
    bp = jnp.dot(ohp, bond8_ref[...], preferred_element_type=_f32)
    ear = (dist * br).astype(_bf)
    eap = (dist * bp).astype(_bf)
    t2 = relu(jnp.dot(ear, wc1a_ref[...], preferred_element_type=_f32)
              + jnp.dot(eap, wc1b_ref[...], preferred_element_type=_f32)
              + bc1_ref[...]).astype(_bf)
    ea = jnp.dot(t2, wc2_ref[...], preferred_element_type=_f32) + bc2_ref[...]
    eab = ea.astype(_bf)
    ea_ref[...] = eab
    for wf1, bf1, wf2, bf2, out in (
            (wf11_ref, bf11_ref, wf21_ref, bf21_ref, f1_ref),
            (wf12_ref, bf12_ref, wf22_ref, bf22_ref, f2_ref),
            (wf13_ref, bf13_ref, wf23_ref, bf23_ref, f3_ref)):
        h = relu(jnp.dot(eab, wf1[...], preferred_element_type=_f32)
                 + bf1[...]).astype(_bf)
        f = jnp.dot(h, wf2[...], preferred_element_type=_f32) + bf2[...]
        out[0] = f[:, :HF]
        out[1] = f[:, HF:]


def _full(shape):
    return pl.BlockSpec(shape, lambda i: (0,) * len(shape))


def _edge_features(el, etr, etp, bond8, params):
    grid = (E // EB,)
    eb_spec = pl.BlockSpec((EB, 1), lambda i: (i, 0))
    ea_spec = pl.BlockSpec((EB, H), lambda i: (i, 0))
    f_spec = pl.BlockSpec((2, EB, HF), lambda i: (0, i, 0))
    lys = params['layers']
    return pl.pallas_call(
        _edge_feat_body,
        grid=grid,
        in_specs=[eb_spec, eb_spec, eb_spec, _full((8, H)),
                  _full((1, H)), _full((1, H)), _full((H, H)), _full((1, H)),
                  _full((H, H)), _full((H, H)), _full((1, H)), _full((H, H)), _full((1, H))]
                 + [_full((H, H)), _full((1, H)), _full((H, H)), _full((1, H))] * 3,
        out_specs=[ea_spec, f_spec, f_spec, f_spec],
        out_shape=[jax.ShapeDtypeStruct((E, H), _bf)]
                  + [jax.ShapeDtypeStruct((2, E, HF), _f32)] * 3,
    )(el, etr, etp, bond8.astype(_bf),
      params['We1'], params['be1'].reshape(1, H),
      params['We2'].astype(_bf), params['be2'].reshape(1, H),
      params['Wc1'][:H].astype(_bf), params['Wc1'][H:].astype(_bf),
      params['bc1'].reshape(1, H),
      params['Wc2'].astype(_bf), params['bc2'].reshape(1, H),
      lys[0]['Wf1'].astype(_bf), lys[0]['bf1'].reshape(1, H),
      lys[0]['Wf2'].astype(_bf), lys[0]['bf2'].reshape(1, H),
      lys[1]['Wf1'].astype(_bf), lys[1]['bf1'].reshape(1, H),
      lys[1]['Wf2'].astype(_bf), lys[1]['bf2'].reshape(1, H),
      lys[2]['Wf1'].astype(_bf), lys[2]['bf1'].reshape(1, H),
      lys[2]['Wf2'].astype(_bf), lys[2]['bf2'].reshape(1, H))


_SC_MESH = plsc.VectorSubcoreMesh(core_axis_name="c", subcore_axis_name="s")
NCH = PER_TILE // CH


def _adjust_vec(buf, b, delta):
    for u in range(CH // 16):
        sl = pl.ds(u * 16, 16)
        buf[b, sl] = buf[b, sl] + delta


@functools.partial(
    pl.kernel,
    out_type=jax.ShapeDtypeStruct((2 * NPAD, HF), _f32),
    mesh=_SC_MESH,
    scratch_types=[
        pltpu.VMEM((2, CH), jnp.int32),
        pltpu.VMEM((2, CH), jnp.int32),
        pltpu.VMEM((2, CH, HF), _f32),
        pltpu.VMEM((2, CH, HF), _f32),
        pltpu.VMEM_SHARED((NPAD, HF), _f32),
        pltpu.SemaphoreType.DMA,
        pltpu.SemaphoreType.DMA,
        pltpu.SemaphoreType.DMA,
        pltpu.SemaphoreType.DMA,
        pltpu.SemaphoreType.DMA,
        pltpu.SemaphoreType.DMA,
        pltpu.SemaphoreType.DMA,
        pltpu.SemaphoreType.DMA,
    ],
)
def _sc_message_pass(xw_hbm, filt_hbm, row_hbm, col_hbm, out_hbm,
                     rowb, colb, gath, filtv, acc,
                     sr0, sr1, sc0, sc1, sg0, sg1, sf0, sf1):
    c = lax.axis_index("c")
    s = lax.axis_index("s")
    cN = c * N
    sri = (sr0, sr1)
    sci = (sc0, sc1)
    sg = (sg0, sg1)
    sf = (sf0, sf1)
    ebase = s * PER_TILE
    fbase = s * PER_TILE

    def zb(j, carry):
        for u in range(HF // 16):
            gath[0, j, pl.ds(u * 16, 16)] = jnp.zeros((16,), _f32)
        return carry

    lax.fori_loop(0, CH, zb, 0, unroll=False)
    for k in range(NROWS // CH):
        off = pl.multiple_of(s * NROWS + k * CH, 8)
        pltpu.sync_copy(gath.at[0], acc.at[pl.ds(off, CH)])
    plsc.subcore_barrier()

    def issue_idx(k, b):
        off = pl.multiple_of(ebase + k * CH, 8)
        pltpu.async_copy(row_hbm.at[pl.ds(off, CH)], rowb.at[b], sri[b])
        pltpu.async_copy(col_hbm.at[pl.ds(off, CH)], colb.at[b], sci[b])

    def issue_gather(k, b):
        pltpu.make_async_copy(row_hbm.at[pl.ds(0, CH)], rowb.at[b], sri[b]).wait()
        pltpu.make_async_copy(col_hbm.at[pl.ds(0, CH)], colb.at[b], sci[b]).wait()
        _adjust_vec(rowb, b, cN)
        pltpu.async_copy(xw_hbm.at[rowb.at[b]], gath.at[b], sg[b])
        pltpu.async_copy(filt_hbm.at[c, pl.ds(fbase + k * CH, CH)],
                         filtv.at[b], sf[b])

    def step(k, b):
        pltpu.make_async_copy(xw_hbm.at[pl.ds(0, CH)], gath.at[b], sg[b]).wait()
        pltpu.make_async_copy(filt_hbm.at[0, pl.ds(0, CH)], filtv.at[b], sf[b]).wait()

        @pl.when(k + 1 < NCH)
        def _():
            issue_gather(k + 1, 1 - b)

        @plsc.parallel_loop(0, CH, 1, unroll=4)
        def _mulrow(j):
            for u in range(HF // 16):
                sl = pl.ds(u * 16, 16)
                gath[b, j, sl] = gath[b, j, sl] * filtv[b, j, sl]
        pltpu.sync_copy(gath.at[b], acc.at[colb.at[b]], add=True)

        @pl.when(k + 2 < NCH)
        def _():
            issue_idx(k + 2, b)

    issue_idx(0, 0)
    issue_idx(1, 1)
    issue_gather(0, 0)

    def pair(i, carry):
        step(2 * i, 0)
        step(2 * i + 1, 1)
        return carry

    lax.fori_loop(0, NCH // 2, pair, 0, unroll=False)
    step(NCH - 1, 0)

    plsc.subcore_barrier()
    src_off = pl.multiple_of(s * NROWS, 8)
    dst_off = pl.multiple_of(c * NPAD + s * NROWS, 8)
    pltpu.sync_copy(acc.at[pl.ds(src_off, NROWS)],
                    out_hbm.at[pl.ds(dst_off, NROWS)])


@functools.partial(
    pl.kernel,
    out_type=jax.ShapeDtypeStruct((2, E, HF), _f32),
    mesh=_SC_MESH,
    scratch_types=[
        pltpu.VMEM((2, CH), jnp.int32),
        pltpu.VMEM((2, CH), jnp.int32),
        pltpu.VMEM((2, CH, HF), _f32),
        pltpu.VMEM((2, CH, HF), _f32),
        pltpu.SemaphoreType.DMA,
        pltpu.SemaphoreType.DMA,
        pltpu.SemaphoreType.DMA,
        pltpu.SemaphoreType.DMA,
        pltpu.SemaphoreType.DMA,
        pltpu.SemaphoreType.DMA,
    ],
)
def _sc_pair_sum(x2_hbm, row_hbm, col_hbm, out_hbm,
                 rowb, colb, g1, g2, sr0, sr1, sc0, sc1, sg0, sg1):
    c = lax.axis_index("c")
    s = lax.axis_index("s")
    cN = c * N
    sri = (sr0, sr1)
    sci = (sc0, sc1)
    sg = (sg0, sg1)
    ebase = s * PER_TILE
    obase = s * PER_TILE

    def issue_idx(k, b):
        off = pl.multiple_of(ebase + k * CH, 8)
        pltpu.async_copy(row_hbm.at[pl.ds(off, CH)], rowb.at[b], sri[b])
        pltpu.async_copy(col_hbm.at[pl.ds(off, CH)], colb.at[b], sci[b])

    def issue_gather(k, b):
        pltpu.make_async_copy(row_hbm.at[pl.ds(0, CH)], rowb.at[b], sri[b]).wait()
        pltpu.make_async_copy(col_hbm.at[pl.ds(0, CH)], colb.at[b], sci[b]).wait()
        _adjust_vec(rowb, b, cN)
        _adjust_vec(colb, b, cN)
        pltpu.async_copy(x2_hbm.at[rowb.at[b]], g1.at[b], sg[b])
        pltpu.async_copy(x2_hbm.at[colb.at[b]], g2.at[b], sg[b])

    def step(k, b):
        pltpu.make_async_copy(x2_hbm.at[pl.ds(0, CH)], g1.at[b], sg[b]).wait()
        pltpu.make_async_copy(x2_hbm.at[pl.ds(0, CH)], g2.at[b], sg[b]).wait()

        @pl.when(k + 1 < NCH)
        def _():
            issue_gather(k + 1, 1 - b)

        @plsc.parallel_loop(0, CH, 1, unroll=4)
        def _addrow(j):
            for u in range(HF // 16):
                sl = pl.ds(u * 16, 16)
                g1[b, j, sl] = g1[b, j, sl] + g2[b, j, sl]
        pltpu.sync_copy(g1.at[b], out_hbm.at[c, pl.ds(obase + k * CH, CH)])

        @pl.when(k + 2 < NCH)
        def _():
            issue_idx(k + 2, b)

    issue_idx(0, 0)
    issue_idx(1, 1)
    issue_gather(0, 0)

    def pair(i, carry):
        step(2 * i, 0)
        step(2 * i + 1, 1)
        return carry

    lax.fori_loop(0, NCH // 2, pair, 0, unroll=False)
    step(NCH - 1, 0)


EPW = E // 32


@functools.partial(
    pl.kernel,
    out_type=jax.ShapeDtypeStruct((E,), _f32),
    mesh=_SC_MESH,
    compiler_params=pltpu.CompilerParams(needs_layout_passes=False),
    scratch_types=[
        pltpu.VMEM((N * 8,), _f32),
        pltpu.VMEM((EPW,), jnp.int32),
        pltpu.VMEM((EPW,), jnp.int32),
        pltpu.VMEM((EPW,), _f32),
    ],
)
def _sc_edge_dist(pos_hbm, row_hbm, col_hbm, out_hbm, posv, rowv, colv, outv):
    c = lax.axis_index("c")
    s = lax.axis_index("s")
    w = s * 2 + c
    base = pl.multiple_of(w * EPW, 8)
    pltpu.sync_copy(pos_hbm, posv)
    pltpu.sync_copy(row_hbm.at[pl.ds(base, EPW)], rowv)
    pltpu.sync_copy(col_hbm.at[pl.ds(base, EPW)], colv)

    def grp_at(off):
        sl = pl.ds(off, 16)
        rv = rowv[sl] * 8
        cv = colv[sl] * 8
        acc = jnp.zeros((16,), _f32)
        for k in range(3):
            a = plsc.load_gather(posv, [rv + k])
            bvals = plsc.load_gather(posv, [cv + k])
            dk = a - bvals
            acc = acc + dk * dk
        outv[sl] = acc

    def grp(g, cr):
        grp_at(g * 16)
        return cr

    lax.fori_loop(0, EPW // 16, grp, 0, unroll=4)
    grp_at(EPW - 16)
    pltpu.sync_copy(outv, out_hbm.at[pl.ds(base, EPW)])


NBLK = 2000


def _node_init_body(at_ref, rf_ref, pf_ref, aemb_ref, waf_ref, wl1_ref,
                    x_ref, xw_ref):
    at = at_ref[...]
    oh_at = (lax.broadcasted_iota(jnp.int32, (NBLK, 104), 1) == at).astype(_f32)
    a_emb = jnp.dot(oh_at, aemb_ref[...], preferred_element_type=_f32)
    iot = lax.broadcasted_iota(jnp.int32, (NBLK, 16), 1)

    def feat(ref):
        r = ref[...]
        acc = jnp.zeros((NBLK, HF), _f32)
        for j in range(3):
            oh = (iot == r[:, j:j + 1]).astype(_f32)
            acc = acc + jnp.dot(oh, waf_ref[pl.ds(j * 16, 16), :],
                                preferred_element_type=_f32)
        return acc

    af_r = feat(rf_ref)
    af_p = feat(pf_ref)
    x = jnp.concatenate([a_emb + af_r, af_p - af_r], axis=-1)
    x_ref[...] = x
    proj = jnp.dot(x, wl1_ref[...], preferred_element_type=_f32)
    xw_ref[0] = proj[:, :HF]
    xw_ref[1] = proj[:, HF:]


def _node_init(atom_type, r_feat, p_feat, aemb_pad, waf_pad, wl1):
    grid = (N // NBLK,)
    return pl.pallas_call(
        _node_init_body,
        grid=grid,
        in_specs=[pl.BlockSpec((NBLK, 1), lambda i: (i, 0)),
                  pl.BlockSpec((NBLK, 3), lambda i: (i, 0)),
                  pl.BlockSpec((NBLK, 3), lambda i: (i, 0)),
                  _full((104, HF)), _full((48, HF)), _full((H, H))],
        out_specs=[pl.BlockSpec((NBLK, H), lambda i: (i, 0)),
                   pl.BlockSpec((2, NBLK, HF), lambda i: (0, i, 0))],
        out_shape=[jax.ShapeDtypeStruct((N, H), _f32),
                   jax.ShapeDtypeStruct((2, N, HF), _f32)],
    )(atom_type, r_feat, p_feat, aemb_pad, waf_pad, wl1)


def _layer_update_body(x_ref, agg_ref, wl2_ref, bl2_ref, wl1n_ref,
                       xn_ref, xw_ref):
    agg = jnp.concatenate([agg_ref[0], agg_ref[1]], axis=-1)
    xn = (x_ref[...] + jnp.dot(jax.nn.relu(agg), wl2_ref[...],
                               preferred_element_type=_f32) + bl2_ref[...])
    xn_ref[...] = xn
    proj = jnp.dot(xn, wl1n_ref[...], preferred_element_type=_f32)
    xw_ref[0] = proj[:, :HF]
    xw_ref[1] = proj[:, HF:]


def _layer_update_last_body(x_ref, agg_ref, wl2_ref, bl2_ref, x2_ref):
    agg = jnp.concatenate([agg_ref[0], agg_ref[1]], axis=-1)
    xn = (x_ref[...] + jnp.dot(jax.nn.relu(agg), wl2_ref[...],
                               preferred_element_type=_f32) + bl2_ref[...])
    x2_ref[0] = xn[:, :HF]
    x2_ref[1] = xn[:, HF:]


def _layer_update(x, agg2, wl2, bl2, wl1n):
    grid = (N // NBLK,)
    agg3 = agg2.reshape(2, NPAD, HF)
    return pl.pallas_call(
        _layer_update_body,
        grid=grid,
        in_specs=[pl.BlockSpec((NBLK, H), lambda i: (i, 0)),
                  pl.BlockSpec((2, NBLK, HF), lambda i: (0, i, 0)),
                  _full((H, H)), _full((1, H)), _full((H, H))],
        out_specs=[pl.BlockSpec((NBLK, H), lambda i: (i, 0)),
                   pl.BlockSpec((2, NBLK, HF), lambda i: (0, i, 0))],
        out_shape=[jax.ShapeDtypeStruct((N, H), _f32),
                   jax.ShapeDtypeStruct((2, N, HF), _f32)],
    )(x, agg3, wl2, bl2.reshape(1, H), wl1n)


def _layer_update_last(x, agg2, wl2, bl2):
    grid = (N // NBLK,)
    agg3 = agg2.reshape(2, NPAD, HF)
    return pl.pallas_call(
        _layer_update_last_body,
        grid=grid,
        in_specs=[pl.BlockSpec((NBLK, H), lambda i: (i, 0)),
                  pl.BlockSpec((2, NBLK, HF), lambda i: (0, i, 0)),
                  _full((H, H)), _full((1, H))],
        out_specs=pl.BlockSpec((2, NBLK, HF), lambda i: (0, i, 0)),
        out_shape=jax.ShapeDtypeStruct((2, N, HF), _f32),
    )(x, agg3, wl2, bl2.reshape(1, H))


def _final_body(hsum_ref, ea_ref, mask_ref,
                wg1a_ref, wg1b_ref, bg1_ref, wg2_ref, bg2_ref, wg3_ref, bg3_ref,
                out_ref):
    relu = jax.nn.relu
    hs = jnp.concatenate([hsum_ref[0], hsum_ref[1]], axis=-1).astype(_bf)
    h1 = relu(jnp.dot(hs, wg1a_ref[...], preferred_element_type=_f32)
              + jnp.dot(ea_ref[...], wg1b_ref[...], preferred_element_type=_f32)
              + bg1_ref[...]).astype(_bf)
    h2 = relu(jnp.dot(h1, wg2_ref[...], preferred_element_type=_f32) + bg2_ref[...])
    o = jnp.dot(h2, wg3_ref[...], preferred_element_type=_f32) + bg3_ref[...]
    out_ref[...] = o * mask_ref[...]


def _final_mlp(hsum2, ea, mask, params):
    grid = (E // EB,)
    hs_spec = pl.BlockSpec((2, EB, HF), lambda i: (0, i, 0))
    big = pl.BlockSpec((EB, H), lambda i: (i, 0))
    one = pl.BlockSpec((EB, 1), lambda i: (i, 0))
    return pl.pallas_call(
        _final_body,
        grid=grid,
        in_specs=[hs_spec, big, one,
                  _full((H, H)), _full((H, H)), _full((1, H)),
                  _full((H, HF)), _full((1, HF)), _full((HF, 1)), _full((1, 1))],
        out_specs=one,
        out_shape=jax.ShapeDtypeStruct((E, 1), _f32),
    )(hsum2, ea, mask,
      params['Wg1'][:H].astype(_bf), params['Wg1'][H:].astype(_bf),
      params['bg1'].reshape(1, H),
      params['Wg2'].astype(_bf), params['bg2'].reshape(1, HF),
      params['Wg3'], params['bg3'].reshape(1, 1))


def kernel(atom_type, r_feat, p_feat, pos, bond_index, bond_type, batch, params):
    lys = params['layers']
    row = bond_index[0].astype(jnp.int32)
    col = bond_index[1].astype(jnp.int32)
    pos_pad = jnp.pad(pos, ((0, 0), (0, 5))).reshape(-1)
    d2 = _sc_edge_dist(pos_pad, row, col)

    bt = bond_type.astype(jnp.int32)
    etr = (bt // NB).reshape(E, 1)
    etp = (bt % NB).reshape(E, 1)
    bond8 = params['bond_emb'][:8]

    ea, f1, f2, f3 = _edge_features(d2.reshape(E, 1), etr, etp, bond8, params)

    aemb_pad = jnp.pad(params['atom_emb'], ((0, 4), (0, 0)))
    waf_pad = jnp.pad(params['W_af'].reshape(3, 10, HF),
                      ((0, 0), (0, 6), (0, 0))).reshape(48, HF)
    x, xw2 = _node_init(atom_type.reshape(N, 1).astype(jnp.int32),
                        r_feat.astype(jnp.int32), p_feat.astype(jnp.int32),
                        aemb_pad, waf_pad, lys[0]['Wl1'])

    for l, filt in enumerate((f1, f2, f3)):
        agg2 = _sc_message_pass(xw2.reshape(2 * N, HF), filt, row, col)
        if l < 2:
            x, xw2 = _layer_update(x, agg2, lys[l]['Wl2'], lys[l]['bl2'],
                                   lys[l + 1]['Wl1'])
        else:
            x2 = _layer_update_last(x, agg2, lys[l]['Wl2'], lys[l]['bl2'])

    hsum2 = _sc_pair_sum(x2.reshape(2 * N, HF), row, col)
    mask = (row < col).astype(_f32).reshape(E, 1)
    return _final_mlp(hsum2, ea, mask, params)
